# Initial kernel scaffold; baseline (speedup 1.0000x reference)
import jax
import jax.numpy as jnp
from jax.experimental import pallas as pl


def kernel(x, edge_index, W_gat, att_src, att_dst, b_gat, W_sage_l, b_sage_l, W_sage_r, W_out, b_out):
    raise NotImplementedError("write your pallas kernel here")


# speedup vs baseline: 5.6459x; 5.6459x over previous
import functools

import jax
import jax.numpy as jnp
import numpy as np
from jax import lax
from jax.experimental import pallas as pl
from jax.experimental.pallas import tpu as pltpu
from jax.experimental.pallas import tpu_sc as plsc

DP = 144
NC = 2
NS = 16
EB = 80
ZR = 25
DR = 125


def _sc_segment_sum(xpad, src, dst, n):
    e = src.shape[0]
    ept = e // (NC * NS)
    nb = ept // EB
    rows_per_tile = n // NS
    nz = rows_per_tile // ZR
    nd = rows_per_tile // DR

    mesh = plsc.VectorSubcoreMesh(core_axis_name="c", subcore_axis_name="s")

    @functools.partial(
        pl.kernel,
        mesh=mesh,
        compiler_params=pltpu.CompilerParams(use_tc_tiling_on_sc=False),
        out_type=jax.ShapeDtypeStruct((NC, n, DP), jnp.float32),
        scratch_types=[
            pltpu.VMEM((EB,), jnp.int32),
            pltpu.VMEM((EB,), jnp.int32),
            pltpu.VMEM((EB, DP), jnp.float32),
            pltpu.VMEM((ZR, DP), jnp.float32),
            pltpu.VMEM((DR, DP), jnp.float32),
            pltpu.VMEM_SHARED((n, DP), jnp.float32),
            pltpu.SemaphoreType.DMA,
        ],
    )
    def body(xpad_hbm, src_hbm, dst_hbm, out_hbm,
             src_v, dst_v, rows_v, zbuf, dbuf, acc_sh, sem):
        c = lax.axis_index("c")
        s = lax.axis_index("s")

        for r in range(ZR):
            for j in range(DP // 16):
                zbuf[r, pl.ds(j * 16, 16)] = jnp.zeros((16,), jnp.float32)
        row0 = s * rows_per_tile
        for k in range(nz):
            pltpu.sync_copy(zbuf, acc_sh.at[pl.ds(row0 + k * ZR, ZR), :])
        plsc.subcore_barrier()

        base = (s * NC + c) * jnp.int32(ept)

        @pl.loop(jnp.int32(0), jnp.int32(nb))
        def step(j):
            off = base + j * jnp.int32(EB)
            pltpu.sync_copy(src_hbm.at[pl.ds(off, EB)], src_v)
            pltpu.sync_copy(dst_hbm.at[pl.ds(off, EB)], dst_v)
            pltpu.async_copy(xpad_hbm.at[src_v], rows_v, sem).wait()
            pltpu.sync_copy(rows_v, acc_sh.at[dst_v], add=True)
        plsc.subcore_barrier()

        for k in range(nd):
            r0 = row0 + k * DR
            pltpu.sync_copy(acc_sh.at[pl.ds(r0, DR), :], dbuf)
            pltpu.sync_copy(dbuf, out_hbm.at[c, pl.ds(r0, DR), :])

    return body(xpad, src, dst)


def _dense_body(x_ref, accp_ref, a_ref, b_ref, r1_ref, bias_ref, wo_ref,
                oh_ref, bout_ref, out_ref):
    x = x_ref[...]
    acc = accp_ref[0] + accp_ref[1]
    deg = jnp.sum(acc * oh_ref[...], axis=1, keepdims=True)
    degc = jnp.maximum(deg, 1.0)
    ind = jnp.minimum(deg, 1.0)
    aggx = acc[:, :128] / degc
    pre = (jnp.dot(aggx, a_ref[...], preferred_element_type=jnp.float32)
           + jnp.dot(x, b_ref[...], preferred_element_type=jnp.float32)
           + ind * r1_ref[...] + bias_ref[...])
    hv = jnp.maximum(pre, 0.0)
    out_ref[...] = jnp.sum(hv * wo_ref[...], axis=1, keepdims=True) + bout_ref[...]


def kernel(x, edge_index, W_gat, att_src, att_dst, b_gat,
           W_sage_l, b_sage_l, W_sage_r, W_out, b_out):
    n, d = x.shape
    h = W_sage_l.shape[0]
    pd = 256

    x = x.astype(jnp.float32)
    src = edge_index[0].astype(jnp.int32)
    dst = edge_index[1].astype(jnp.int32)

    xpad = jnp.concatenate(
        [x, jnp.ones((n, 1), jnp.float32), jnp.zeros((n, DP - d - 1), jnp.float32)],
        axis=1)

    accp = _sc_segment_sum(xpad, src, dst, n)

    f32 = jnp.float32
    wl = W_sage_l.astype(f32)
    wr = W_sage_r.astype(f32)
    a_p = jnp.pad(wl[:d, :], ((0, 0), (0, pd - h)))
    b_p = jnp.pad(wr[:d, :], ((0, 0), (0, pd - h)))
    r1_p = jnp.pad(wl[d:d + 1, :], ((0, 0), (0, pd - h)))
    bias_p = jnp.pad(b_sage_l.astype(f32)[None, :] + wr[d:d + 1, :],
                     ((0, 0), (0, pd - h)))
    wo_p = jnp.pad(W_out.astype(f32)[:, 0][None, :], ((0, 0), (0, pd - h)))
    onehot = jnp.zeros((1, DP), f32).at[0, d].set(1.0)
    bout = b_out.astype(f32).reshape(1, 1)

    z = np.int32(0)
    blk = 400
    grid = n // blk
    out = pl.pallas_call(
        _dense_body,
        grid=(grid,),
        in_specs=[
            pl.BlockSpec((blk, d), lambda i: (i, z)),
            pl.BlockSpec((NC, blk, DP), lambda i: (z, i, z)),
            pl.BlockSpec((d, pd), lambda i: (z, z)),
            pl.BlockSpec((d, pd), lambda i: (z, z)),
            pl.BlockSpec((1, pd), lambda i: (z, z)),
            pl.BlockSpec((1, pd), lambda i: (z, z)),
            pl.BlockSpec((1, pd), lambda i: (z, z)),
            pl.BlockSpec((1, DP), lambda i: (z, z)),
            pl.BlockSpec((1, 1), lambda i: (z, z)),
        ],
        out_specs=pl.BlockSpec((blk, 1), lambda i: (i, z)),
        out_shape=jax.ShapeDtypeStruct((n, 1), jnp.float32),
    )(x, accp, a_p, b_p, r1_p, bias_p, wo_p, onehot, bout)
    return out


# speedup vs baseline: 6.1827x; 1.0951x over previous
import functools

import jax
import jax.numpy as jnp
import numpy as np
from jax import lax
from jax.experimental import pallas as pl
from jax.experimental.pallas import tpu as pltpu
from jax.experimental.pallas import tpu_sc as plsc

NC = 2
NS = 16
EB = 128
CH = 16


def _sc_segment_sum(x, src3, dst3, n, npad, nb):
    nch = -(-nb // CH)
    rpt = npad // NS

    mesh = plsc.VectorSubcoreMesh(core_axis_name="c", subcore_axis_name="s")

    @functools.partial(
        pl.kernel,
        mesh=mesh,
        compiler_params=pltpu.CompilerParams(use_tc_tiling_on_sc=False),
        out_type=[
            jax.ShapeDtypeStruct((NC, npad, 128), jnp.float32),
            jax.ShapeDtypeStruct((NC, npad), jnp.float32),
        ],
        scratch_types=[
            pltpu.VMEM((EB, 128), jnp.float32),
            pltpu.VMEM((EB, 128), jnp.float32),
            pltpu.VMEM((CH, EB), jnp.int32),
            pltpu.VMEM((CH, EB), jnp.int32),
            pltpu.VMEM((CH, EB), jnp.int32),
            pltpu.VMEM((CH, EB), jnp.int32),
            pltpu.VMEM((EB,), jnp.float32),
            pltpu.VMEM((npad // NS,), jnp.float32),
            pltpu.VMEM_SHARED((npad, 128), jnp.float32),
            pltpu.VMEM_SHARED((npad,), jnp.float32),
            pltpu.SemaphoreType.DMA,
            pltpu.SemaphoreType.DMA,
            pltpu.SemaphoreType.DMA,
            pltpu.SemaphoreType.DMA,
            pltpu.SemaphoreType.DMA,
            pltpu.SemaphoreType.DMA,
            pltpu.SemaphoreType.DMA,
        ],
    )
    def body(x_hbm, src_hbm, dst_hbm, acc_out, deg_out,
             b0, b1, sic0, sic1, dic0, dic1, vone, degb,
             acc_sh, deg_sh,
             g0, g1, s0, s1, i0, i1, dsem):
        c = lax.axis_index("c")
        s = lax.axis_index("s")
        t = s * NC + c
        bufs = [b0, b1]
        sics = [sic0, sic1]
        dics = [dic0, dic1]
        gsems = [g0, g1]
        ssems = [s0, s1]
        isems = [i0, i1]

        zv = jnp.zeros((16,), jnp.float32)
        onev = zv + jnp.float32(1)
        for r in range(EB):
            for j in range(8):
                b0[r, pl.ds(j * 16, 16)] = zv
        for j in range(EB // 16):
            vone[pl.ds(j * 16, 16)] = onev
        for j in range(rpt // 16):
            degb[pl.ds(j * 16, 16)] = zv
        row0 = s * rpt
        for k in range(rpt // EB):
            pltpu.sync_copy(b0, acc_sh.at[pl.ds(row0 + k * EB, EB), :])
        pltpu.sync_copy(degb, deg_sh.at[pl.ds(row0, rpt)])
        plsc.subcore_barrier()

        def load_chunk(q):
            qb = q % 2
            return (pltpu.async_copy(src_hbm.at[t, pl.ds(np.int32(q * CH), CH), :],
                                     sics[qb], isems[qb]),
                    pltpu.async_copy(dst_hbm.at[t, pl.ds(np.int32(q * CH), CH), :],
                                     dics[qb], isems[qb]))

        def g(i):
            return pltpu.async_copy(x_hbm.at[sics[(i // CH) % 2].at[np.int32(i % CH)]],
                                    bufs[i % 2], gsems[i % 2])

        for i in range(nb):
            b = i % 2
            q = i // CH
            if i % CH == 0:
                cds = load_chunk(q)
                cds[0].wait()
                cds[1].wait()
            dic = dics[q % 2]
            g(i).wait()
            pltpu.async_copy(bufs[b], acc_sh.at[dic.at[np.int32(i % CH)]],
                             ssems[b], add=True).wait()
            pltpu.async_copy(vone, deg_sh.at[dic.at[np.int32(i % CH)]],
                             dsem, add=True).wait()
        plsc.subcore_barrier()

        for k in range(rpt // EB):
            r0 = row0 + k * EB
            pltpu.sync_copy(acc_sh.at[pl.ds(r0, EB), :], bufs[k % 2])
            pltpu.sync_copy(bufs[k % 2], acc_out.at[c, pl.ds(r0, EB), :])
        pltpu.sync_copy(deg_sh.at[pl.ds(row0, rpt)], degb)
        pltpu.sync_copy(degb, deg_out.at[c, pl.ds(row0, rpt)])

    return body(x, src3, dst3)


def _dense_body(x_ref, accp_ref, deg0_ref, deg1_ref, a_ref, b_ref, r1_ref,
                bias_ref, wo_ref, bout_ref, out_ref):
    x = x_ref[...]
    acc = accp_ref[0] + accp_ref[1]
    deg = deg0_ref[...] + deg1_ref[...]
    degc = jnp.maximum(deg, 1.0)
    ind = jnp.minimum(deg, 1.0)
    aggx = acc / degc
    pre = (jnp.dot(aggx, a_ref[...], preferred_element_type=jnp.float32)
           + jnp.dot(x, b_ref[...], preferred_element_type=jnp.float32)
           + ind * r1_ref[...] + bias_ref[...])
    hv = jnp.maximum(pre, 0.0)
    out_ref[...] = jnp.sum(hv * wo_ref[...], axis=1, keepdims=True) + bout_ref[...]


def kernel(x, edge_index, W_gat, att_src, att_dst, b_gat,
           W_sage_l, b_sage_l, W_sage_r, W_out, b_out):
    n, d = x.shape
    e = edge_index.shape[1]
    h = W_sage_l.shape[0]
    pd = 256
    f32 = jnp.float32

    nt = NC * NS
    ept = e // nt
    nb = -(-ept // EB)
    nbh = (-(-nb // CH)) * CH
    npad = NS * (-(-(n + 1) // (NS * EB)) * EB)

    x = x.astype(f32)
    src2 = edge_index[0].astype(jnp.int32).reshape(nt, ept)
    dst2 = edge_index[1].astype(jnp.int32).reshape(nt, ept)
    pad = nbh * EB - ept
    src3 = jnp.pad(src2, ((0, 0), (0, pad))).reshape(nt, nbh, EB)
    dst3 = jnp.pad(dst2, ((0, 0), (0, pad)),
                   constant_values=n).reshape(nt, nbh, EB)

    accp, degp = _sc_segment_sum(x, src3, dst3, n, npad, nb)
    deg0 = degp[0].reshape(npad, 1)
    deg1 = degp[1].reshape(npad, 1)

    wl = W_sage_l.astype(f32)
    wr = W_sage_r.astype(f32)
    a_p = jnp.pad(wl[:d, :], ((0, 0), (0, pd - h)))
    b_p = jnp.pad(wr[:d, :], ((0, 0), (0, pd - h)))
    r1_p = jnp.pad(wl[d:d + 1, :], ((0, 0), (0, pd - h)))
    bias_p = jnp.pad(b_sage_l.astype(f32)[None, :] + wr[d:d + 1, :],
                     ((0, 0), (0, pd - h)))
    wo_p = jnp.pad(W_out.astype(f32)[:, 0][None, :], ((0, 0), (0, pd - h)))
    bout = b_out.astype(f32).reshape(1, 1)
    xp = jnp.pad(x, ((0, npad - n), (0, 0)))

    z = np.int32(0)
    blk = 512
    grid = npad // blk
    out = pl.pallas_call(
        _dense_body,
        grid=(grid,),
        in_specs=[
            pl.BlockSpec((blk, d), lambda i: (i, z)),
            pl.BlockSpec((NC, blk, 128), lambda i: (z, i, z)),
            pl.BlockSpec((blk, 1), lambda i: (i, z)),
            pl.BlockSpec((blk, 1), lambda i: (i, z)),
            pl.BlockSpec((d, pd), lambda i: (z, z)),
            pl.BlockSpec((d, pd), lambda i: (z, z)),
            pl.BlockSpec((1, pd), lambda i: (z, z)),
            pl.BlockSpec((1, pd), lambda i: (z, z)),
            pl.BlockSpec((1, pd), lambda i: (z, z)),
            pl.BlockSpec((1, 1), lambda i: (z, z)),
        ],
        out_specs=pl.BlockSpec((blk, 1), lambda i: (i, z)),
        out_shape=jax.ShapeDtypeStruct((npad, 1), jnp.float32),
    )(xp, accp, deg0, deg1, a_p, b_p, r1_p, bias_p, wo_p, bout)
    return out[:n]


# speedup vs baseline: 9.0108x; 1.4574x over previous
import functools

import jax
import jax.numpy as jnp
import numpy as np
from jax import lax
from jax.experimental import pallas as pl
from jax.experimental.pallas import tpu as pltpu
from jax.experimental.pallas import tpu_sc as plsc

NC = 2
NS = 16
EB = 64
CH = 16


def _sc_segment_sum(x, src3, dst3, n, npad, nb):
    nch = -(-nb // CH)
    rpt = npad // NS

    mesh = plsc.VectorSubcoreMesh(core_axis_name="c", subcore_axis_name="s")

    @functools.partial(
        pl.kernel,
        mesh=mesh,
        compiler_params=pltpu.CompilerParams(use_tc_tiling_on_sc=False),
        out_type=[
            jax.ShapeDtypeStruct((NC, npad, 128), jnp.float32),
            jax.ShapeDtypeStruct((NC, npad), jnp.float32),
        ],
        scratch_types=[
            pltpu.VMEM((EB, 128), jnp.float32),
            pltpu.VMEM((EB, 128), jnp.float32),
            pltpu.VMEM((EB, 128), jnp.float32),
            pltpu.VMEM((EB, 128), jnp.float32),
            pltpu.VMEM((CH, EB), jnp.int32),
            pltpu.VMEM((CH, EB), jnp.int32),
            pltpu.VMEM((CH, EB), jnp.int32),
            pltpu.VMEM((CH, EB), jnp.int32),
            pltpu.VMEM((EB,), jnp.float32),
            pltpu.VMEM((npad // NS,), jnp.float32),
            pltpu.VMEM_SHARED((npad, 128), jnp.float32),
            pltpu.VMEM_SHARED((npad,), jnp.float32),
            pltpu.SemaphoreType.DMA,
            pltpu.SemaphoreType.DMA,
            pltpu.SemaphoreType.DMA,
            pltpu.SemaphoreType.DMA,
            pltpu.SemaphoreType.DMA,
            pltpu.SemaphoreType.DMA,
            pltpu.SemaphoreType.DMA,
            pltpu.SemaphoreType.DMA,
            pltpu.SemaphoreType.DMA,
            pltpu.SemaphoreType.DMA,
            pltpu.SemaphoreType.DMA,
        ],
    )
    def body(x_hbm, src_hbm, dst_hbm, acc_out, deg_out,
             b0, b1, b2, b3, sic0, sic1, dic0, dic1, vone, degb,
             acc_sh, deg_sh,
             g0, g1, g2, g3, s0, s1, s2, s3, i0, i1, dsem):
        c = lax.axis_index("c")
        s = lax.axis_index("s")
        t = s * NC + c
        bufs = [b0, b1, b2, b3]
        sics = [sic0, sic1]
        dics = [dic0, dic1]
        gsems = [g0, g1, g2, g3]
        ssems = [s0, s1, s2, s3]
        isems = [i0, i1]

        zv = jnp.zeros((16,), jnp.float32)
        onev = zv + jnp.float32(1)
        for r in range(EB):
            for j in range(8):
                b0[r, pl.ds(j * 16, 16)] = zv
        for j in range(EB // 16):
            vone[pl.ds(j * 16, 16)] = onev
        for j in range(rpt // 16):
            degb[pl.ds(j * 16, 16)] = zv
        row0 = s * rpt
        for k in range(rpt // EB):
            pltpu.sync_copy(b0, acc_sh.at[pl.ds(row0 + k * EB, EB), :])
        pltpu.sync_copy(degb, deg_sh.at[pl.ds(row0, rpt)])
        plsc.subcore_barrier()

        def load_chunk(q):
            qb = q % 2
            return (pltpu.async_copy(src_hbm.at[t, pl.ds(np.int32(q * CH), CH), :],
                                     sics[qb], isems[qb]),
                    pltpu.async_copy(dst_hbm.at[t, pl.ds(np.int32(q * CH), CH), :],
                                     dics[qb], isems[qb]))

        def g(i):
            return pltpu.async_copy(x_hbm.at[sics[(i // CH) % 2].at[np.int32(i % CH)]],
                                    bufs[i % 4], gsems[i % 4])

        icd = [None] * nch
        icd[0] = load_chunk(0)
        if nch > 1:
            icd[1] = load_chunk(1)
        icd[0][0].wait()
        icd[0][1].wait()
        gd = [None] * nb
        sd = [None] * nb
        dd = [None] * nb
        gd[0] = g(0)
        if nb > 1:
            gd[1] = g(1)
        for i in range(nb):
            b = i % 4
            dic = dics[(i // CH) % 2]
            gd[i].wait()
            sd[i] = pltpu.async_copy(bufs[b], acc_sh.at[dic.at[np.int32(i % CH)]],
                                     ssems[b], add=True)
            sd[i].wait()
            dd[i] = pltpu.async_copy(vone, deg_sh.at[dic.at[np.int32(i % CH)]],
                                     dsem, add=True)
            dd[i].wait()
            ni = i + 2
            if ni < nb:
                if ni % CH == 0 and ni // CH < nch:
                    icd[ni // CH][0].wait()
                    icd[ni // CH][1].wait()
                gd[ni] = g(ni)
            if i % CH == 1 and 2 <= i // CH + 1 < nch:
                icd[i // CH + 1] = load_chunk(i // CH + 1)

        plsc.subcore_barrier()

        for k in range(rpt // EB):
            r0 = row0 + k * EB
            pltpu.sync_copy(acc_sh.at[pl.ds(r0, EB), :], bufs[k % 4])
            pltpu.sync_copy(bufs[k % 4], acc_out.at[c, pl.ds(r0, EB), :])
        pltpu.sync_copy(deg_sh.at[pl.ds(row0, rpt)], degb)
        pltpu.sync_copy(degb, deg_out.at[c, pl.ds(row0, rpt)])

    return body(x, src3, dst3)


def _dense_body(x_ref, accp_ref, deg0_ref, deg1_ref, a_ref, b_ref, r1_ref,
                bias_ref, wo_ref, bout_ref, out_ref):
    x = x_ref[...]
    acc = accp_ref[0] + accp_ref[1]
    deg = deg0_ref[...] + deg1_ref[...]
    degc = jnp.maximum(deg, 1.0)
    ind = jnp.minimum(deg, 1.0)
    aggx = acc / degc
    pre = (jnp.dot(aggx, a_ref[...], preferred_element_type=jnp.float32)
           + jnp.dot(x, b_ref[...], preferred_element_type=jnp.float32)
           + ind * r1_ref[...] + bias_ref[...])
    hv = jnp.maximum(pre, 0.0)
    out_ref[...] = jnp.sum(hv * wo_ref[...], axis=1, keepdims=True) + bout_ref[...]


def kernel(x, edge_index, W_gat, att_src, att_dst, b_gat,
           W_sage_l, b_sage_l, W_sage_r, W_out, b_out):
    n, d = x.shape
    e = edge_index.shape[1]
    h = W_sage_l.shape[0]
    pd = 256
    f32 = jnp.float32

    nt = NC * NS
    ept = e // nt
    nb = -(-ept // EB)
    nbh = (-(-nb // CH)) * CH
    npad = NS * (-(-(n + 1) // (NS * EB)) * EB)

    x = x.astype(f32)
    src2 = edge_index[0].astype(jnp.int32).reshape(nt, ept)
    dst2 = edge_index[1].astype(jnp.int32).reshape(nt, ept)
    pad = nbh * EB - ept
    src3 = jnp.pad(src2, ((0, 0), (0, pad))).reshape(nt, nbh, EB)
    dst3 = jnp.pad(dst2, ((0, 0), (0, pad)),
                   constant_values=n).reshape(nt, nbh, EB)

    accp, degp = _sc_segment_sum(x, src3, dst3, n, npad, nb)
    deg0 = degp[0].reshape(npad, 1)
    deg1 = degp[1].reshape(npad, 1)

    wl = W_sage_l.astype(f32)
    wr = W_sage_r.astype(f32)
    a_p = jnp.pad(wl[:d, :], ((0, 0), (0, pd - h)))
    b_p = jnp.pad(wr[:d, :], ((0, 0), (0, pd - h)))
    r1_p = jnp.pad(wl[d:d + 1, :], ((0, 0), (0, pd - h)))
    bias_p = jnp.pad(b_sage_l.astype(f32)[None, :] + wr[d:d + 1, :],
                     ((0, 0), (0, pd - h)))
    wo_p = jnp.pad(W_out.astype(f32)[:, 0][None, :], ((0, 0), (0, pd - h)))
    bout = b_out.astype(f32).reshape(1, 1)
    xp = jnp.pad(x, ((0, npad - n), (0, 0)))

    z = np.int32(0)
    blk = 512
    grid = npad // blk
    out = pl.pallas_call(
        _dense_body,
        grid=(grid,),
        in_specs=[
            pl.BlockSpec((blk, d), lambda i: (i, z)),
            pl.BlockSpec((NC, blk, 128), lambda i: (z, i, z)),
            pl.BlockSpec((blk, 1), lambda i: (i, z)),
            pl.BlockSpec((blk, 1), lambda i: (i, z)),
            pl.BlockSpec((d, pd), lambda i: (z, z)),
            pl.BlockSpec((d, pd), lambda i: (z, z)),
            pl.BlockSpec((1, pd), lambda i: (z, z)),
            pl.BlockSpec((1, pd), lambda i: (z, z)),
            pl.BlockSpec((1, pd), lambda i: (z, z)),
            pl.BlockSpec((1, 1), lambda i: (z, z)),
        ],
        out_specs=pl.BlockSpec((blk, 1), lambda i: (i, z)),
        out_shape=jax.ShapeDtypeStruct((npad, 1), jnp.float32),
    )(xp, accp, deg0, deg1, a_p, b_p, r1_p, bias_p, wo_p, bout)
    return out[:n]
